# Initial kernel scaffold; baseline (speedup 1.0000x reference)
import jax
import jax.numpy as jnp
from jax.experimental import pallas as pl


def kernel(r0, m0, pos, tetra, tetra_face, face_vertex, face_tetra):
    raise NotImplementedError("write your pallas kernel here")


# speedup vs baseline: 3.7893x; 3.7893x over previous
import jax
import jax.numpy as jnp
import numpy as np
from jax import lax
from jax.experimental import pallas as pl
from jax.experimental.pallas import tpu as pltpu

N_RAYS = 2048
N_STEPS = 16
N_SAMPLES = 64
NEAR = 0.0
FAR = 1.0
TWO_PI = np.float32(2.0 * np.pi)
TCHUNK = 512
BIG = 2 ** 30


def _bdot3(ax, ay, az, bx, by, bz):
    return (ax * bx + ay * by) + az * bz


def _tables(r0, m0, pos, tetra, tetra_face, face_vertex, face_tetra):
    num_tetra = tetra.shape[1]
    n_index = -0.1 * jnp.linalg.norm(pos, axis=1) + 1.1
    coords = jnp.transpose(pos[tetra], (1, 2, 0))
    kmat = jnp.linalg.inv(jnp.concatenate(
        [jnp.ones((num_tetra, 1, 4), coords.dtype), coords], axis=1))
    ab = jnp.matmul(jnp.transpose(kmat, (0, 2, 1)), n_index[tetra].T[..., None])
    a_all = ab[:, 0, 0]
    b_all = ab[:, 1:, 0]
    bn = jnp.linalg.norm(b_all, axis=1)
    n_all = b_all / bn[:, None]
    anb = a_all / bn
    v0 = pos[tetra[0]]
    emat = jnp.stack([pos[tetra[1]] - v0, pos[tetra[2]] - v0,
                      pos[tetra[3]] - v0], axis=2)
    ort = jnp.linalg.inv(emat)
    cvec = jnp.einsum('tij,tj->ti', ort, v0)
    loc = jnp.concatenate([ort.reshape(num_tetra, 9), cvec], axis=1)
    i_ = pos[face_vertex[:, 0]]
    j_ = pos[face_vertex[:, 1]]
    k_ = pos[face_vertex[:, 2]]
    ML0 = (j_[:, 1] - i_[:, 1]) * (k_[:, 2] - i_[:, 2]) - (k_[:, 1] - i_[:, 1]) * (j_[:, 2] - i_[:, 2])
    ML1 = (j_[:, 2] - i_[:, 2]) * (k_[:, 0] - i_[:, 0]) - (k_[:, 2] - i_[:, 2]) * (j_[:, 0] - i_[:, 0])
    ML2 = (j_[:, 0] - i_[:, 0]) * (k_[:, 1] - i_[:, 1]) - (k_[:, 0] - i_[:, 0]) * (j_[:, 1] - i_[:, 1])
    ML = jnp.stack([ML0, ML1, ML2], axis=1)
    QL = -jnp.sum(i_ * ML, axis=1)
    MLt = ML[tetra_face].reshape(num_tetra, 12)
    QLt = QL[tetra_face]
    ft = face_tetra[tetra_face]
    t_ids = jnp.arange(num_tetra, dtype=ft.dtype)[:, None]
    sel = jnp.argmax(ft != t_ids[..., None], axis=2)
    nxt = jnp.take_along_axis(ft, sel[..., None], axis=2)[..., 0]
    ft0 = ft[:, :, 0]
    tab = jnp.concatenate(
        [n_all, anb[:, None], MLt, QLt,
         nxt.astype(jnp.float32), ft0.astype(jnp.float32)], axis=1)
    tab = jnp.where(jnp.isfinite(tab), tab, 0.0)
    return loc, tab.T


def _body(rT_ref, mT_ref, loc_ref, tab_ref, z_ref, out_ref,
          dd_ref, rx_ref, ry_ref, rz_ref):
    T = loc_ref.shape[0]
    n_chunks = T // TCHUNK
    rx = rT_ref[0:1, :]
    ry = rT_ref[1:2, :]
    rz = rT_ref[2:3, :]

    def loc_step(c, acc):
        ch = loc_ref[pl.ds(c * TCHUNK, TCHUNK), :]
        np0 = _bdot3(ch[:, 0:1], ch[:, 1:2], ch[:, 2:3], rx, ry, rz) - ch[:, 9:10]
        np1 = _bdot3(ch[:, 3:4], ch[:, 4:5], ch[:, 5:6], rx, ry, rz) - ch[:, 10:11]
        np2 = _bdot3(ch[:, 6:7], ch[:, 7:8], ch[:, 8:9], rx, ry, rz) - ch[:, 11:12]
        s = (np0 + np1) + np2
        val = ((np0 >= 0.0) & (np1 >= 0.0) & (np2 >= 0.0)
               & (np0 <= 1.0) & (np1 <= 1.0) & (np2 <= 1.0) & (s <= 1.0))
        tids = lax.broadcasted_iota(jnp.int32, val.shape, 0) + c * TCHUNK
        cand = jnp.min(jnp.where(val, tids, BIG), axis=0, keepdims=True)
        return jnp.minimum(acc, cand)

    acc0 = jnp.full((1, N_RAYS), BIG, jnp.int32)
    hit = lax.fori_loop(0, n_chunks, loc_step, acc0)
    idx0 = jnp.where(hit >= BIG, jnp.int32(-1), hit)

    dd_ref[0:1, :] = jnp.zeros((1, N_RAYS), jnp.float32)
    rx_ref[0:1, :] = rx
    ry_ref[0:1, :] = ry
    rz_ref[0:1, :] = rz

    def step(k, carry):
        idx, px, py, pz, mx, my, mz, dcum = carry
        wrapped = jnp.where(idx < 0, idx + T, idx)

        def gat(c, acc):
            oh = (lax.broadcasted_iota(jnp.int32, (TCHUNK, N_RAYS), 0)
                  + c * TCHUNK == wrapped).astype(jnp.float32)
            part = lax.dot_general(
                tab_ref[:, pl.ds(c * TCHUNK, TCHUNK)], oh,
                (((1,), (0,)), ((), ())),
                preferred_element_type=jnp.float32)
            return acc + part

        g = lax.fori_loop(0, n_chunks, gat, jnp.zeros((28, N_RAYS), jnp.float32))
        nx = g[0:1, :]
        ny = g[1:2, :]
        nz = g[2:3, :]
        anb = g[3:4, :]
        mnx = my * nz - mz * ny
        mny = mz * nx - mx * nz
        mnz = mx * ny - my * nx
        mg = jnp.sqrt(_bdot3(mnx, mny, mnz, mnx, mny, mnz))
        qx = mnx / mg
        qy = mny / mg
        qz = mnz / mg
        nqx = ny * qz - nz * qy
        nqy = nz * qx - nx * qz
        nqz = nx * qy - ny * qx
        mn_dot = _bdot3(mx, my, mz, nx, ny, nz)
        mnq = _bdot3(mx, my, mz, nqx, nqy, nqz)
        coef = _bdot3(px, py, pz, nx, ny, nz) + anb
        rcx = px - coef * (nx - mn_dot * nqx / mnq)
        rcy = py - coef * (ny - mn_dot * nqy / mnq)
        rcz = pz - coef * (nz - mn_dot * nqz / mnq)
        Rx = rcx - px
        Ry = rcy - py
        Rz = rcz - pz
        Rn = jnp.sqrt(_bdot3(Rx, Ry, Rz, Rx, Ry, Rz))

        best = None
        bidx = None
        for f in range(4):
            mlx = g[4 + 3 * f:5 + 3 * f, :]
            mly = g[5 + 3 * f:6 + 3 * f, :]
            mlz = g[6 + 3 * f:7 + 3 * f, :]
            ql = g[16 + f:17 + f, :]
            c1 = -_bdot3(mlx, mly, mlz, Rx, Ry, Rz)
            c2 = Rn * _bdot3(mlx, mly, mlz, mx, my, mz)
            c3 = _bdot3(mlx, mly, mlz, rcx, rcy, rcz) + ql
            disc = jnp.sqrt((c1 * c1 + c2 * c2) - c3 * c3)
            den = c1 - c3
            one = jnp.ones_like(c1)
            phi1 = jnp.mod(2.0 * lax.atan2((c2 + disc) / den, one), TWO_PI)
            phi2 = jnp.mod(2.0 * lax.atan2((c2 - disc) / den, one), TWO_PI)
            phif = jnp.minimum(phi1, phi2)
            phif = jnp.where(jnp.isnan(phif), jnp.float32(10.0), phif)
            if f == 0:
                best = phif
                bidx = jnp.zeros_like(phif, jnp.int32)
            else:
                lt = phif < best
                best = jnp.where(lt, phif, best)
                bidx = jnp.where(lt, jnp.int32(f), bidx)
        phiE = best + best / 100.0
        cph = jnp.cos(phiE)
        sph = jnp.sin(phiE)
        rex = rcx - cph * Rx + Rn * sph * mx
        rey = rcy - cph * Ry + Rn * sph * my
        rez = rcz - cph * Rz + Rn * sph * mz
        mex = cph * mx + sph / Rn * Rx
        mey = cph * my + sph / Rn * Ry
        mez = cph * mz + sph / Rn * Rz
        nxtf = g[20:21, :]
        ft0f = g[24:25, :]
        for f in range(1, 4):
            m_f = bidx == f
            nxtf = jnp.where(m_f, g[20 + f:21 + f, :], nxtf)
            ft0f = jnp.where(m_f, g[24 + f:25 + f, :], ft0f)
        nxt = jnp.where(idx < 0, ft0f, nxtf).astype(jnp.int32)
        dx = px - rex
        dy = py - rey
        dz = pz - rez
        dist = jnp.sqrt(_bdot3(dx, dy, dz, dx, dy, dz))
        dnew = dcum + dist
        dd_ref[pl.ds(k + 1, 1), :] = dnew
        rx_ref[pl.ds(k + 1, 1), :] = rex
        ry_ref[pl.ds(k + 1, 1), :] = rey
        rz_ref[pl.ds(k + 1, 1), :] = rez
        return (nxt, rex, rey, rez, mex, mey, mez, dnew)

    mT = mT_ref[...]
    lax.fori_loop(
        0, N_STEPS, step,
        (idx0, rx, ry, rz, mT[0:1, :], mT[1:2, :], mT[2:3, :],
         jnp.zeros((1, N_RAYS), jnp.float32)))

    z = z_ref[...]
    cnt = jnp.zeros((N_SAMPLES, N_RAYS), jnp.int32)
    for k in range(N_STEPS + 1):
        cnt = cnt + (dd_ref[k:k + 1, :] <= z).astype(jnp.int32)
    sidx = jnp.clip(cnt - 1, 0, N_STEPS - 1)
    d0 = jnp.zeros((N_SAMPLES, N_RAYS), jnp.float32)
    d1 = jnp.zeros_like(d0)
    x0 = jnp.zeros_like(d0)
    x1 = jnp.zeros_like(d0)
    y0 = jnp.zeros_like(d0)
    y1 = jnp.zeros_like(d0)
    z0 = jnp.zeros_like(d0)
    z1 = jnp.zeros_like(d0)
    for k in range(N_STEPS):
        msk = (sidx == k).astype(jnp.float32)
        d0 = d0 + msk * dd_ref[k:k + 1, :]
        d1 = d1 + msk * dd_ref[k + 1:k + 2, :]
        x0 = x0 + msk * rx_ref[k:k + 1, :]
        x1 = x1 + msk * rx_ref[k + 1:k + 2, :]
        y0 = y0 + msk * ry_ref[k:k + 1, :]
        y1 = y1 + msk * ry_ref[k + 1:k + 2, :]
        z0 = z0 + msk * rz_ref[k:k + 1, :]
        z1 = z1 + msk * rz_ref[k + 1:k + 2, :]
    delta = d1 - d0
    denom = jnp.where(jnp.abs(delta) > 1e-12, delta, jnp.float32(1.0))
    frac = (z - d0) / denom
    out_ref[0, :, :] = x0 + frac * (x1 - x0)
    out_ref[1, :, :] = y0 + frac * (y1 - y0)
    out_ref[2, :, :] = z0 + frac * (z1 - z0)


def kernel(r0, m0, pos, tetra, tetra_face, face_vertex, face_tetra):
    loc, tabT = _tables(r0, m0, pos, tetra, tetra_face, face_vertex,
                        face_tetra)
    t_vals = jnp.linspace(0.1, 1.0, N_SAMPLES).astype(r0.dtype)
    z = (NEAR * (1.0 - t_vals) + FAR * t_vals)[:, None]
    out = pl.pallas_call(
        _body,
        out_shape=jax.ShapeDtypeStruct((3, N_SAMPLES, N_RAYS), jnp.float32),
        scratch_shapes=[
            pltpu.VMEM((N_STEPS + 8, N_RAYS), jnp.float32),
            pltpu.VMEM((N_STEPS + 8, N_RAYS), jnp.float32),
            pltpu.VMEM((N_STEPS + 8, N_RAYS), jnp.float32),
            pltpu.VMEM((N_STEPS + 8, N_RAYS), jnp.float32),
        ],
    )(r0.T, m0.T, loc, tabT, z)
    return jnp.transpose(out, (2, 1, 0))


# speedup vs baseline: 3.7898x; 1.0001x over previous
import jax
import jax.numpy as jnp
import numpy as np
from jax import lax
from jax.experimental import pallas as pl
from jax.experimental.pallas import tpu as pltpu

N_RAYS = 2048
N_STEPS = 16
N_SAMPLES = 64
NEAR = 0.0
FAR = 1.0
TWO_PI = np.float32(2.0 * np.pi)
TCHUNK = 512
BIG = 2 ** 30


def _bdot3(ax, ay, az, bx, by, bz):
    return (ax * bx + ay * by) + az * bz


def _tables(r0, m0, pos, tetra, tetra_face, face_vertex, face_tetra):
    num_tetra = tetra.shape[1]
    n_index = -0.1 * jnp.linalg.norm(pos, axis=1) + 1.1
    coords = jnp.transpose(pos[tetra], (1, 2, 0))
    kmat = jnp.linalg.inv(jnp.concatenate(
        [jnp.ones((num_tetra, 1, 4), coords.dtype), coords], axis=1))
    ab = jnp.matmul(jnp.transpose(kmat, (0, 2, 1)), n_index[tetra].T[..., None])
    a_all = ab[:, 0, 0]
    b_all = ab[:, 1:, 0]
    bn = jnp.linalg.norm(b_all, axis=1)
    n_all = b_all / bn[:, None]
    anb = a_all / bn
    v0 = pos[tetra[0]]
    emat = jnp.stack([pos[tetra[1]] - v0, pos[tetra[2]] - v0,
                      pos[tetra[3]] - v0], axis=2)
    ort = jnp.linalg.inv(emat)
    cvec = jnp.einsum('tij,tj->ti', ort, v0)
    loc = jnp.concatenate([ort.reshape(num_tetra, 9), cvec], axis=1)
    i_ = pos[face_vertex[:, 0]]
    j_ = pos[face_vertex[:, 1]]
    k_ = pos[face_vertex[:, 2]]
    ML0 = (j_[:, 1] - i_[:, 1]) * (k_[:, 2] - i_[:, 2]) - (k_[:, 1] - i_[:, 1]) * (j_[:, 2] - i_[:, 2])
    ML1 = (j_[:, 2] - i_[:, 2]) * (k_[:, 0] - i_[:, 0]) - (k_[:, 2] - i_[:, 2]) * (j_[:, 0] - i_[:, 0])
    ML2 = (j_[:, 0] - i_[:, 0]) * (k_[:, 1] - i_[:, 1]) - (k_[:, 0] - i_[:, 0]) * (j_[:, 1] - i_[:, 1])
    ML = jnp.stack([ML0, ML1, ML2], axis=1)
    QL = -jnp.sum(i_ * ML, axis=1)
    MLt = jnp.transpose(ML[tetra_face], (0, 2, 1)).reshape(num_tetra, 12)
    QLt = QL[tetra_face]
    ft = face_tetra[tetra_face]
    t_ids = jnp.arange(num_tetra, dtype=ft.dtype)[:, None]
    sel = jnp.argmax(ft != t_ids[..., None], axis=2)
    nxt = jnp.take_along_axis(ft, sel[..., None], axis=2)[..., 0]
    ft0 = ft[:, :, 0]
    tab = jnp.concatenate(
        [n_all, anb[:, None], MLt, QLt,
         nxt.astype(jnp.float32), ft0.astype(jnp.float32)], axis=1)
    tab = jnp.where(jnp.isfinite(tab), tab, 0.0)
    return loc, tab.T


def _body(rT_ref, mT_ref, loc_ref, tab_ref, z_ref, out_ref,
          dd_ref, rx_ref, ry_ref, rz_ref):
    T = loc_ref.shape[0]
    n_chunks = T // TCHUNK
    rx = rT_ref[0:1, :]
    ry = rT_ref[1:2, :]
    rz = rT_ref[2:3, :]

    def loc_step(c, acc):
        ch = loc_ref[pl.ds(c * TCHUNK, TCHUNK), :]
        np0 = _bdot3(ch[:, 0:1], ch[:, 1:2], ch[:, 2:3], rx, ry, rz) - ch[:, 9:10]
        np1 = _bdot3(ch[:, 3:4], ch[:, 4:5], ch[:, 5:6], rx, ry, rz) - ch[:, 10:11]
        np2 = _bdot3(ch[:, 6:7], ch[:, 7:8], ch[:, 8:9], rx, ry, rz) - ch[:, 11:12]
        s = (np0 + np1) + np2
        val = ((np0 >= 0.0) & (np1 >= 0.0) & (np2 >= 0.0)
               & (np0 <= 1.0) & (np1 <= 1.0) & (np2 <= 1.0) & (s <= 1.0))
        tids = lax.broadcasted_iota(jnp.int32, val.shape, 0) + c * TCHUNK
        cand = jnp.min(jnp.where(val, tids, BIG), axis=0, keepdims=True)
        return jnp.minimum(acc, cand)

    acc0 = jnp.full((1, N_RAYS), BIG, jnp.int32)
    hit = lax.fori_loop(0, n_chunks, loc_step, acc0)
    idx0 = jnp.where(hit >= BIG, jnp.int32(-1), hit)

    dd_ref[0:1, :] = jnp.zeros((1, N_RAYS), jnp.float32)
    rx_ref[0:1, :] = rx
    ry_ref[0:1, :] = ry
    rz_ref[0:1, :] = rz

    def step(k, carry):
        idx, px, py, pz, mx, my, mz, dcum = carry
        wrapped = jnp.where(idx < 0, idx + T, idx)

        def gat(c, acc):
            oh = (lax.broadcasted_iota(jnp.int32, (TCHUNK, N_RAYS), 0)
                  + c * TCHUNK == wrapped).astype(jnp.float32)
            part = lax.dot_general(
                tab_ref[:, pl.ds(c * TCHUNK, TCHUNK)], oh,
                (((1,), (0,)), ((), ())),
                preferred_element_type=jnp.float32)
            return acc + part

        g = lax.fori_loop(0, n_chunks, gat, jnp.zeros((28, N_RAYS), jnp.float32))
        nx = g[0:1, :]
        ny = g[1:2, :]
        nz = g[2:3, :]
        anb = g[3:4, :]
        mnx = my * nz - mz * ny
        mny = mz * nx - mx * nz
        mnz = mx * ny - my * nx
        mg = jnp.sqrt(_bdot3(mnx, mny, mnz, mnx, mny, mnz))
        qx = mnx / mg
        qy = mny / mg
        qz = mnz / mg
        nqx = ny * qz - nz * qy
        nqy = nz * qx - nx * qz
        nqz = nx * qy - ny * qx
        mn_dot = _bdot3(mx, my, mz, nx, ny, nz)
        mnq = _bdot3(mx, my, mz, nqx, nqy, nqz)
        coef = _bdot3(px, py, pz, nx, ny, nz) + anb
        rcx = px - coef * (nx - mn_dot * nqx / mnq)
        rcy = py - coef * (ny - mn_dot * nqy / mnq)
        rcz = pz - coef * (nz - mn_dot * nqz / mnq)
        Rx = rcx - px
        Ry = rcy - py
        Rz = rcz - pz
        Rn = jnp.sqrt(_bdot3(Rx, Ry, Rz, Rx, Ry, Rz))

        mlx = g[4:8, :]
        mly = g[8:12, :]
        mlz = g[12:16, :]
        ql = g[16:20, :]
        c1 = -_bdot3(mlx, mly, mlz, Rx, Ry, Rz)
        c2 = Rn * _bdot3(mlx, mly, mlz, mx, my, mz)
        c3 = _bdot3(mlx, mly, mlz, rcx, rcy, rcz) + ql
        disc = jnp.sqrt((c1 * c1 + c2 * c2) - c3 * c3)
        den = c1 - c3
        u = jnp.concatenate([(c2 + disc) / den, (c2 - disc) / den], axis=0)
        phi = jnp.mod(2.0 * lax.atan2(u, jnp.ones_like(u)), TWO_PI)
        phif = jnp.minimum(phi[0:4, :], phi[4:8, :])
        phif = jnp.where(jnp.isnan(phif), jnp.float32(10.0), phif)
        best = jnp.min(phif, axis=0, keepdims=True)
        fio = lax.broadcasted_iota(jnp.int32, (4, N_RAYS), 0)
        bidx = jnp.min(jnp.where(phif == best, fio, jnp.int32(4)),
                       axis=0, keepdims=True)
        phiE = best + best / 100.0
        cph = jnp.cos(phiE)
        sph = jnp.sin(phiE)
        rex = rcx - cph * Rx + Rn * sph * mx
        rey = rcy - cph * Ry + Rn * sph * my
        rez = rcz - cph * Rz + Rn * sph * mz
        mex = cph * mx + sph / Rn * Rx
        mey = cph * my + sph / Rn * Ry
        mez = cph * mz + sph / Rn * Rz
        oh4 = (fio == bidx).astype(jnp.float32)
        nxtf = jnp.sum(oh4 * g[20:24, :], axis=0, keepdims=True)
        ft0f = jnp.sum(oh4 * g[24:28, :], axis=0, keepdims=True)
        nxt = jnp.where(idx < 0, ft0f, nxtf).astype(jnp.int32)
        dx = px - rex
        dy = py - rey
        dz = pz - rez
        dist = jnp.sqrt(_bdot3(dx, dy, dz, dx, dy, dz))
        dnew = dcum + dist
        dd_ref[pl.ds(k + 1, 1), :] = dnew
        rx_ref[pl.ds(k + 1, 1), :] = rex
        ry_ref[pl.ds(k + 1, 1), :] = rey
        rz_ref[pl.ds(k + 1, 1), :] = rez
        return (nxt, rex, rey, rez, mex, mey, mez, dnew)

    mT = mT_ref[...]
    lax.fori_loop(
        0, N_STEPS, step,
        (idx0, rx, ry, rz, mT[0:1, :], mT[1:2, :], mT[2:3, :],
         jnp.zeros((1, N_RAYS), jnp.float32)))

    z = z_ref[...]
    cnt = jnp.zeros((N_SAMPLES, N_RAYS), jnp.int32)
    for k in range(N_STEPS + 1):
        cnt = cnt + (dd_ref[k:k + 1, :] <= z).astype(jnp.int32)
    sidx = jnp.clip(cnt - 1, 0, N_STEPS - 1)
    d0 = jnp.zeros((N_SAMPLES, N_RAYS), jnp.float32)
    d1 = jnp.zeros_like(d0)
    x0 = jnp.zeros_like(d0)
    x1 = jnp.zeros_like(d0)
    y0 = jnp.zeros_like(d0)
    y1 = jnp.zeros_like(d0)
    z0 = jnp.zeros_like(d0)
    z1 = jnp.zeros_like(d0)
    for k in range(N_STEPS):
        msk = (sidx == k).astype(jnp.float32)
        d0 = d0 + msk * dd_ref[k:k + 1, :]
        d1 = d1 + msk * dd_ref[k + 1:k + 2, :]
        x0 = x0 + msk * rx_ref[k:k + 1, :]
        x1 = x1 + msk * rx_ref[k + 1:k + 2, :]
        y0 = y0 + msk * ry_ref[k:k + 1, :]
        y1 = y1 + msk * ry_ref[k + 1:k + 2, :]
        z0 = z0 + msk * rz_ref[k:k + 1, :]
        z1 = z1 + msk * rz_ref[k + 1:k + 2, :]
    delta = d1 - d0
    denom = jnp.where(jnp.abs(delta) > 1e-12, delta, jnp.float32(1.0))
    frac = (z - d0) / denom
    out_ref[0, :, :] = x0 + frac * (x1 - x0)
    out_ref[1, :, :] = y0 + frac * (y1 - y0)
    out_ref[2, :, :] = z0 + frac * (z1 - z0)


def kernel(r0, m0, pos, tetra, tetra_face, face_vertex, face_tetra):
    loc, tabT = _tables(r0, m0, pos, tetra, tetra_face, face_vertex,
                        face_tetra)
    t_vals = jnp.linspace(0.1, 1.0, N_SAMPLES).astype(r0.dtype)
    z = (NEAR * (1.0 - t_vals) + FAR * t_vals)[:, None]
    out = pl.pallas_call(
        _body,
        out_shape=jax.ShapeDtypeStruct((3, N_SAMPLES, N_RAYS), jnp.float32),
        scratch_shapes=[
            pltpu.VMEM((N_STEPS + 8, N_RAYS), jnp.float32),
            pltpu.VMEM((N_STEPS + 8, N_RAYS), jnp.float32),
            pltpu.VMEM((N_STEPS + 8, N_RAYS), jnp.float32),
            pltpu.VMEM((N_STEPS + 8, N_RAYS), jnp.float32),
        ],
    )(r0.T, m0.T, loc, tabT, z)
    return jnp.transpose(out, (2, 1, 0))


# speedup vs baseline: 46.6945x; 12.3211x over previous
import jax
import jax.numpy as jnp
import numpy as np
from jax import lax
from jax.experimental import pallas as pl
from jax.experimental.pallas import tpu as pltpu

N_RAYS = 2048
N_STEPS = 16
N_SAMPLES = 64
NEAR = 0.0
FAR = 1.0
TWO_PI = np.float32(2.0 * np.pi)
TCHUNK = 512
BIG = 2 ** 30


def _bdot3(ax, ay, az, bx, by, bz):
    return (ax * bx + ay * by) + az * bz


def _tables(r0, m0, pos, tetra, tetra_face, face_vertex, face_tetra):
    num_tetra = tetra.shape[1]
    n_index = -0.1 * jnp.sqrt(jnp.sum(pos * pos, axis=1)) + 1.1
    v0 = pos[tetra[0]]
    v1 = pos[tetra[1]]
    v2 = pos[tetra[2]]
    v3 = pos[tetra[3]]
    e1 = v1 - v0
    e2 = v2 - v0
    e3 = v3 - v0
    c11 = e2[:, 1] * e3[:, 2] - e3[:, 1] * e2[:, 2]
    c12 = e3[:, 0] * e2[:, 2] - e2[:, 0] * e3[:, 2]
    c13 = e2[:, 0] * e3[:, 1] - e3[:, 0] * e2[:, 1]
    c21 = e3[:, 1] * e1[:, 2] - e1[:, 1] * e3[:, 2]
    c22 = e1[:, 0] * e3[:, 2] - e3[:, 0] * e1[:, 2]
    c23 = e3[:, 0] * e1[:, 1] - e1[:, 0] * e3[:, 1]
    c31 = e1[:, 1] * e2[:, 2] - e2[:, 1] * e1[:, 2]
    c32 = e2[:, 0] * e1[:, 2] - e1[:, 0] * e2[:, 2]
    c33 = e1[:, 0] * e2[:, 1] - e2[:, 0] * e1[:, 1]
    det = e1[:, 0] * c11 + e2[:, 0] * c21 + e3[:, 0] * c31
    inv_det = 1.0 / det
    ort = (jnp.stack([c11, c12, c13, c21, c22, c23, c31, c32, c33], axis=1)
           * inv_det[:, None]).reshape(num_tetra, 3, 3)
    nn0 = n_index[tetra[0]]
    dn = jnp.stack([n_index[tetra[1]] - nn0, n_index[tetra[2]] - nn0,
                    n_index[tetra[3]] - nn0], axis=1)
    b_all = jnp.einsum('tij,ti->tj', ort, dn)
    a_all = nn0 - jnp.sum(b_all * v0, axis=1)
    bn = jnp.sqrt(jnp.sum(b_all * b_all, axis=1))
    n_all = b_all / bn[:, None]
    anb = a_all / bn
    cvec = jnp.einsum('tij,tj->ti', ort, v0)
    loc = jnp.concatenate([ort.reshape(num_tetra, 9), cvec], axis=1)
    i_ = pos[face_vertex[:, 0]]
    j_ = pos[face_vertex[:, 1]]
    k_ = pos[face_vertex[:, 2]]
    ML0 = (j_[:, 1] - i_[:, 1]) * (k_[:, 2] - i_[:, 2]) - (k_[:, 1] - i_[:, 1]) * (j_[:, 2] - i_[:, 2])
    ML1 = (j_[:, 2] - i_[:, 2]) * (k_[:, 0] - i_[:, 0]) - (k_[:, 2] - i_[:, 2]) * (j_[:, 0] - i_[:, 0])
    ML2 = (j_[:, 0] - i_[:, 0]) * (k_[:, 1] - i_[:, 1]) - (k_[:, 0] - i_[:, 0]) * (j_[:, 1] - i_[:, 1])
    ML = jnp.stack([ML0, ML1, ML2], axis=1)
    QL = -jnp.sum(i_ * ML, axis=1)
    MLt = jnp.transpose(ML[tetra_face], (0, 2, 1)).reshape(num_tetra, 12)
    QLt = QL[tetra_face]
    ft = face_tetra[tetra_face]
    t_ids = jnp.arange(num_tetra, dtype=ft.dtype)[:, None]
    sel = jnp.argmax(ft != t_ids[..., None], axis=2)
    nxt = jnp.take_along_axis(ft, sel[..., None], axis=2)[..., 0]
    ft0 = ft[:, :, 0]
    tab = jnp.concatenate(
        [n_all, anb[:, None], MLt, QLt,
         nxt.astype(jnp.float32), ft0.astype(jnp.float32)], axis=1)
    tab = jnp.where(jnp.isfinite(tab), tab, 0.0)
    return loc, tab.T


def _body(rT_ref, mT_ref, loc_ref, tab_ref, z_ref, out_ref,
          dd_ref, rx_ref, ry_ref, rz_ref):
    T = loc_ref.shape[0]
    n_chunks = T // TCHUNK
    rx = rT_ref[0:1, :]
    ry = rT_ref[1:2, :]
    rz = rT_ref[2:3, :]

    def loc_step(c, acc):
        ch = loc_ref[pl.ds(c * TCHUNK, TCHUNK), :]
        np0 = _bdot3(ch[:, 0:1], ch[:, 1:2], ch[:, 2:3], rx, ry, rz) - ch[:, 9:10]
        np1 = _bdot3(ch[:, 3:4], ch[:, 4:5], ch[:, 5:6], rx, ry, rz) - ch[:, 10:11]
        np2 = _bdot3(ch[:, 6:7], ch[:, 7:8], ch[:, 8:9], rx, ry, rz) - ch[:, 11:12]
        s = (np0 + np1) + np2
        val = ((np0 >= 0.0) & (np1 >= 0.0) & (np2 >= 0.0)
               & (np0 <= 1.0) & (np1 <= 1.0) & (np2 <= 1.0) & (s <= 1.0))
        tids = lax.broadcasted_iota(jnp.int32, val.shape, 0) + c * TCHUNK
        cand = jnp.min(jnp.where(val, tids, BIG), axis=0, keepdims=True)
        return jnp.minimum(acc, cand)

    acc0 = jnp.full((1, N_RAYS), BIG, jnp.int32)
    hit = lax.fori_loop(0, n_chunks, loc_step, acc0)
    idx0 = jnp.where(hit >= BIG, jnp.int32(-1), hit)

    dd_ref[0:1, :] = jnp.zeros((1, N_RAYS), jnp.float32)
    rx_ref[0:1, :] = rx
    ry_ref[0:1, :] = ry
    rz_ref[0:1, :] = rz

    def step(k, carry):
        idx, px, py, pz, mx, my, mz, dcum = carry
        wrapped = jnp.where(idx < 0, idx + T, idx)

        def gat(c, acc):
            oh = (lax.broadcasted_iota(jnp.int32, (TCHUNK, N_RAYS), 0)
                  + c * TCHUNK == wrapped).astype(jnp.float32)
            part = lax.dot_general(
                tab_ref[:, pl.ds(c * TCHUNK, TCHUNK)], oh,
                (((1,), (0,)), ((), ())),
                preferred_element_type=jnp.float32)
            return acc + part

        g = lax.fori_loop(0, n_chunks, gat, jnp.zeros((28, N_RAYS), jnp.float32))
        nx = g[0:1, :]
        ny = g[1:2, :]
        nz = g[2:3, :]
        anb = g[3:4, :]
        mnx = my * nz - mz * ny
        mny = mz * nx - mx * nz
        mnz = mx * ny - my * nx
        mg = jnp.sqrt(_bdot3(mnx, mny, mnz, mnx, mny, mnz))
        qx = mnx / mg
        qy = mny / mg
        qz = mnz / mg
        nqx = ny * qz - nz * qy
        nqy = nz * qx - nx * qz
        nqz = nx * qy - ny * qx
        mn_dot = _bdot3(mx, my, mz, nx, ny, nz)
        mnq = _bdot3(mx, my, mz, nqx, nqy, nqz)
        coef = _bdot3(px, py, pz, nx, ny, nz) + anb
        rcx = px - coef * (nx - mn_dot * nqx / mnq)
        rcy = py - coef * (ny - mn_dot * nqy / mnq)
        rcz = pz - coef * (nz - mn_dot * nqz / mnq)
        Rx = rcx - px
        Ry = rcy - py
        Rz = rcz - pz
        Rn = jnp.sqrt(_bdot3(Rx, Ry, Rz, Rx, Ry, Rz))

        mlx = g[4:8, :]
        mly = g[8:12, :]
        mlz = g[12:16, :]
        ql = g[16:20, :]
        c1 = -_bdot3(mlx, mly, mlz, Rx, Ry, Rz)
        c2 = Rn * _bdot3(mlx, mly, mlz, mx, my, mz)
        c3 = _bdot3(mlx, mly, mlz, rcx, rcy, rcz) + ql
        disc = jnp.sqrt((c1 * c1 + c2 * c2) - c3 * c3)
        den = c1 - c3
        u = jnp.concatenate([(c2 + disc) / den, (c2 - disc) / den], axis=0)
        phi = jnp.mod(2.0 * lax.atan2(u, jnp.ones_like(u)), TWO_PI)
        phif = jnp.minimum(phi[0:4, :], phi[4:8, :])
        phif = jnp.where(jnp.isnan(phif), jnp.float32(10.0), phif)
        best = jnp.min(phif, axis=0, keepdims=True)
        fio = lax.broadcasted_iota(jnp.int32, (4, N_RAYS), 0)
        bidx = jnp.min(jnp.where(phif == best, fio, jnp.int32(4)),
                       axis=0, keepdims=True)
        phiE = best + best / 100.0
        cph = jnp.cos(phiE)
        sph = jnp.sin(phiE)
        rex = rcx - cph * Rx + Rn * sph * mx
        rey = rcy - cph * Ry + Rn * sph * my
        rez = rcz - cph * Rz + Rn * sph * mz
        mex = cph * mx + sph / Rn * Rx
        mey = cph * my + sph / Rn * Ry
        mez = cph * mz + sph / Rn * Rz
        oh4 = (fio == bidx).astype(jnp.float32)
        nxtf = jnp.sum(oh4 * g[20:24, :], axis=0, keepdims=True)
        ft0f = jnp.sum(oh4 * g[24:28, :], axis=0, keepdims=True)
        nxt = jnp.where(idx < 0, ft0f, nxtf).astype(jnp.int32)
        dx = px - rex
        dy = py - rey
        dz = pz - rez
        dist = jnp.sqrt(_bdot3(dx, dy, dz, dx, dy, dz))
        dnew = dcum + dist
        dd_ref[pl.ds(k + 1, 1), :] = dnew
        rx_ref[pl.ds(k + 1, 1), :] = rex
        ry_ref[pl.ds(k + 1, 1), :] = rey
        rz_ref[pl.ds(k + 1, 1), :] = rez
        return (nxt, rex, rey, rez, mex, mey, mez, dnew)

    mT = mT_ref[...]
    lax.fori_loop(
        0, N_STEPS, step,
        (idx0, rx, ry, rz, mT[0:1, :], mT[1:2, :], mT[2:3, :],
         jnp.zeros((1, N_RAYS), jnp.float32)))

    z = z_ref[...]
    cnt = jnp.zeros((N_SAMPLES, N_RAYS), jnp.int32)
    for k in range(N_STEPS + 1):
        cnt = cnt + (dd_ref[k:k + 1, :] <= z).astype(jnp.int32)
    sidx = jnp.clip(cnt - 1, 0, N_STEPS - 1)
    d0 = jnp.zeros((N_SAMPLES, N_RAYS), jnp.float32)
    d1 = jnp.zeros_like(d0)
    x0 = jnp.zeros_like(d0)
    x1 = jnp.zeros_like(d0)
    y0 = jnp.zeros_like(d0)
    y1 = jnp.zeros_like(d0)
    z0 = jnp.zeros_like(d0)
    z1 = jnp.zeros_like(d0)
    for k in range(N_STEPS):
        msk = (sidx == k).astype(jnp.float32)
        d0 = d0 + msk * dd_ref[k:k + 1, :]
        d1 = d1 + msk * dd_ref[k + 1:k + 2, :]
        x0 = x0 + msk * rx_ref[k:k + 1, :]
        x1 = x1 + msk * rx_ref[k + 1:k + 2, :]
        y0 = y0 + msk * ry_ref[k:k + 1, :]
        y1 = y1 + msk * ry_ref[k + 1:k + 2, :]
        z0 = z0 + msk * rz_ref[k:k + 1, :]
        z1 = z1 + msk * rz_ref[k + 1:k + 2, :]
    delta = d1 - d0
    denom = jnp.where(jnp.abs(delta) > 1e-12, delta, jnp.float32(1.0))
    frac = (z - d0) / denom
    out_ref[0, :, :] = x0 + frac * (x1 - x0)
    out_ref[1, :, :] = y0 + frac * (y1 - y0)
    out_ref[2, :, :] = z0 + frac * (z1 - z0)


def kernel(r0, m0, pos, tetra, tetra_face, face_vertex, face_tetra):
    loc, tabT = _tables(r0, m0, pos, tetra, tetra_face, face_vertex,
                        face_tetra)
    t_vals = jnp.linspace(0.1, 1.0, N_SAMPLES).astype(r0.dtype)
    z = (NEAR * (1.0 - t_vals) + FAR * t_vals)[:, None]
    out = pl.pallas_call(
        _body,
        out_shape=jax.ShapeDtypeStruct((3, N_SAMPLES, N_RAYS), jnp.float32),
        scratch_shapes=[
            pltpu.VMEM((N_STEPS + 8, N_RAYS), jnp.float32),
            pltpu.VMEM((N_STEPS + 8, N_RAYS), jnp.float32),
            pltpu.VMEM((N_STEPS + 8, N_RAYS), jnp.float32),
            pltpu.VMEM((N_STEPS + 8, N_RAYS), jnp.float32),
        ],
    )(r0.T, m0.T, loc, tabT, z)
    return jnp.transpose(out, (2, 1, 0))
